# baseline (speedup 1.0000x reference)
import jax
import jax.numpy as jnp
from jax import lax
from jax.experimental import pallas as pl
from jax.experimental.pallas import tpu as pltpu
from jax.experimental.pallas import tpu_sc as plsc

_B = 16384
_NUM_REG = 10
_NUM_CAT = 26
_VOCAB = 100000
_D = 32
_NCOLS = _NUM_REG + _NUM_CAT

_NC = 2
_NS = 16
_NW = _NC * _NS
_BC = 8192
_HA = 49920
_HB = _VOCAB - _HA
_U = 8


def _body(ainT_hbm, tableT_hbm, regw_hbm, regb_hbm, outT_hbm,
          rowa_v, rowb_v, idx_v, outp_v, regw_v, regb_v,
          sem_a, sem_b, sem_o0, sem_o1):
    c = lax.axis_index("c")
    s = lax.axis_index("s")
    w = s * _NC + c

    pltpu.sync_copy(regw_hbm, regw_v)
    pltpu.sync_copy(regb_hbm, regb_v)

    sem_o = [sem_o0, sem_o1]

    def start_row(i):
        pltpu.async_copy(tableT_hbm.at[i, w, pl.ds(0, _HA)], rowa_v, sem_a)
        pltpu.async_copy(tableT_hbm.at[i, w, pl.ds(_HA, _HB)], rowb_v, sem_b)

    def wait_a():
        pltpu.make_async_copy(
            tableT_hbm.at[0, 0, pl.ds(0, _HA)], rowa_v, sem_a).wait()

    def wait_b():
        pltpu.make_async_copy(
            tableT_hbm.at[0, 0, pl.ds(_HA, _HB)], rowb_v, sem_b).wait()

    def wait_out(h):
        pltpu.make_async_copy(
            outp_v.at[h], outT_hbm.at[0, 0, pl.ds(0, _BC)], sem_o[h]).wait()

    def load_idx(col, h):
        pltpu.sync_copy(ainT_hbm.at[col, pl.ds(h * _BC, _BC)], idx_v)

    def pass_a(h):
        def step(g, _):
            base = g * (16 * _U)
            for u in range(_U):
                sl = pl.ds(base + u * 16, 16)
                iv = idx_v[sl]
                outp_v[h, sl] = plsc.load_gather(
                    rowa_v, [jnp.minimum(iv, _HA - 1)])
            return 0
        lax.fori_loop(0, _BC // (16 * _U), step, 0)

    def pass_b(h):
        def step(g, _):
            base = g * (16 * _U)
            for u in range(_U):
                sl = pl.ds(base + u * 16, 16)
                iv = idx_v[sl]
                gb = plsc.load_gather(
                    rowb_v, [jnp.maximum(iv - _HA, 0)])
                outp_v[h, sl] = jnp.where(iv >= _HA, gb, outp_v[h, sl])
            return 0
        lax.fori_loop(0, _BC // (16 * _U), step, 0)

    start_row(0)

    def row_body(i, _):
        col = _NUM_REG + i
        wait_a()
        for h in range(2):
            load_idx(col, h)

            @pl.when(i > 0)
            def _():
                wait_out(h)
            pass_a(h)

        @pl.when(i + 1 < _NUM_CAT)
        def _():
            pltpu.async_copy(
                tableT_hbm.at[i + 1, w, pl.ds(0, _HA)], rowa_v, sem_a)
        wait_b()
        for h in range(2):
            load_idx(col, h)
            pass_b(h)
            pltpu.async_copy(outp_v.at[h],
                             outT_hbm.at[col - _NUM_REG, w,
                                         pl.ds(h * _BC, _BC)], sem_o[h])

        @pl.when(i + 1 < _NUM_CAT)
        def _():
            pltpu.async_copy(
                tableT_hbm.at[i + 1, w, pl.ds(_HA, _HB)], rowb_v, sem_b)
        return 0
    lax.fori_loop(0, _NUM_CAT, row_body, 0)

    for j in range(_NUM_REG):
        sel = jnp.full((16,), j * _D, jnp.int32) + w
        ws = plsc.load_gather(regw_v, [sel])
        bs = plsc.load_gather(regb_v, [sel])
        for h in range(2):
            load_idx(j, h)
            wait_out(h)

            def rstep(g, _):
                base = g * (16 * _U)
                for u in range(_U):
                    sl = pl.ds(base + u * 16, 16)
                    xf = idx_v[sl].astype(jnp.float32)
                    outp_v[h, sl] = xf * ws + bs
                return 0
            lax.fori_loop(0, _BC // (16 * _U), rstep, 0)
            pltpu.async_copy(outp_v.at[h],
                             outT_hbm.at[_NUM_CAT + j, w,
                                         pl.ds(h * _BC, _BC)], sem_o[h])
    wait_out(0)
    wait_out(1)


_sc_call = pl.kernel(
    _body,
    out_type=jax.ShapeDtypeStruct((_NCOLS, _D, _B), jnp.float32),
    mesh=plsc.VectorSubcoreMesh(core_axis_name="c", subcore_axis_name="s"),
    scratch_types=[
        pltpu.VMEM((_HA,), jnp.float32),
        pltpu.VMEM((_HB,), jnp.float32),
        pltpu.VMEM((_BC,), jnp.int32),
        pltpu.VMEM((2, _BC), jnp.float32),
        pltpu.VMEM((_NUM_REG * _D,), jnp.float32),
        pltpu.VMEM((_NUM_REG * _D,), jnp.float32),
        pltpu.SemaphoreType.DMA,
        pltpu.SemaphoreType.DMA,
        pltpu.SemaphoreType.DMA,
        pltpu.SemaphoreType.DMA,
    ],
    compiler_params=pltpu.CompilerParams(
        needs_layout_passes=False, use_tc_tiling_on_sc=True),
)


@jax.jit
def kernel(all_inputs, emb_tables, reg_W, reg_b):
    ainT = all_inputs.T
    tableT = emb_tables.transpose(0, 2, 1)
    outT = _sc_call(ainT, tableT, reg_W.reshape(-1), reg_b.reshape(-1))
    return outT.transpose(2, 0, 1)


# speedup vs baseline: 3.0880x; 3.0880x over previous
import jax
import jax.numpy as jnp
from jax import lax
from jax.experimental import pallas as pl
from jax.experimental.pallas import tpu as pltpu
from jax.experimental.pallas import tpu_sc as plsc

_B = 16384
_NUM_REG = 10
_NUM_CAT = 26
_VOCAB = 100000
_D = 32
_NCOLS = _NUM_REG + _NUM_CAT

_NC = 2
_NS = 16
_BC = 8192


def _body(ainT_hbm, tableT_hbm, regw_hbm, regb_hbm, outT_hbm,
          row_v, idx_v, out_v, regw_v, regb_v, sem_i, sem_o0, sem_o1):
    w = lax.axis_index("s") * _NC + lax.axis_index("c")

    pltpu.sync_copy(regw_hbm, regw_v)
    pltpu.sync_copy(regb_hbm, regb_v)

    sem_o = [sem_o0, sem_o1]
    oc = [None, None]

    for i in range(_NUM_CAT):
        col = _NUM_REG + i
        ic = pltpu.async_copy(ainT_hbm.at[col, pl.ds(0, _BC)], idx_v, sem_i)
        pltpu.sync_copy(tableT_hbm.at[i, w], row_v)
        for h in range(2):
            if h == 0:
                ic.wait()
            else:
                pltpu.sync_copy(ainT_hbm.at[col, pl.ds(_BC, _BC)], idx_v)
            if oc[h] is not None:
                oc[h].wait()

            @plsc.parallel_loop(0, _BC // 16, unroll=8)
            def _(g):
                sl = pl.ds(g * 16, 16)
                out_v[h, sl] = plsc.load_gather(row_v, [idx_v[sl]])
            oc[h] = pltpu.async_copy(
                out_v.at[h], outT_hbm.at[i, w, pl.ds(h * _BC, _BC)],
                sem_o[h])

    for j in range(_NUM_REG):
        sel = jnp.full((16,), j * _D, jnp.int32) + w
        ws = plsc.load_gather(regw_v, [sel])
        bs = plsc.load_gather(regb_v, [sel])
        for h in range(2):
            pltpu.sync_copy(ainT_hbm.at[j, pl.ds(h * _BC, _BC)], idx_v)
            if oc[h] is not None:
                oc[h].wait()

            @plsc.parallel_loop(0, _BC // 16, unroll=8)
            def _(g):
                sl = pl.ds(g * 16, 16)
                out_v[h, sl] = idx_v[sl].astype(jnp.float32) * ws + bs
            oc[h] = pltpu.async_copy(
                out_v.at[h], outT_hbm.at[_NUM_CAT + j, w,
                                         pl.ds(h * _BC, _BC)], sem_o[h])
    oc[0].wait()
    oc[1].wait()


_sc_call = pl.kernel(
    _body,
    out_type=jax.ShapeDtypeStruct((_NCOLS, _D, _B), jnp.float32),
    mesh=plsc.VectorSubcoreMesh(core_axis_name="c", subcore_axis_name="s"),
    scratch_types=[
        pltpu.VMEM((_VOCAB,), jnp.float32),
        pltpu.VMEM((_BC,), jnp.int32),
        pltpu.VMEM((2, _BC), jnp.float32),
        pltpu.VMEM((_NUM_REG * _D,), jnp.float32),
        pltpu.VMEM((_NUM_REG * _D,), jnp.float32),
        pltpu.SemaphoreType.DMA,
        pltpu.SemaphoreType.DMA,
        pltpu.SemaphoreType.DMA,
    ],
    compiler_params=pltpu.CompilerParams(
        needs_layout_passes=False, use_tc_tiling_on_sc=True),
)


@jax.jit
def kernel(all_inputs, emb_tables, reg_W, reg_b):
    ainT = all_inputs.T
    tableT = emb_tables.transpose(0, 2, 1)
    outT = _sc_call(ainT, tableT, reg_W.reshape(-1), reg_b.reshape(-1))
    return outT.transpose(2, 0, 1)


# speedup vs baseline: 3.2461x; 1.0512x over previous
import jax
import jax.numpy as jnp
from jax import lax
from jax.experimental import pallas as pl
from jax.experimental.pallas import tpu as pltpu
from jax.experimental.pallas import tpu_sc as plsc

_B = 16384
_NUM_REG = 10
_NUM_CAT = 26
_VOCAB = 100000
_D = 32
_NCOLS = _NUM_REG + _NUM_CAT

_NC = 2
_NS = 16
_BC = 8192


def _body(ainT_hbm, tableT_hbm, regw_hbm, regb_hbm, outT_hbm,
          row_v, idx_v, regw_v, regb_v, sem_i0, sem_i1, sem_o0, sem_o1):
    w = lax.axis_index("s") * _NC + lax.axis_index("c")

    pltpu.sync_copy(regw_hbm, regw_v)
    pltpu.sync_copy(regb_hbm, regb_v)

    sem_i = [sem_i0, sem_i1]
    sem_o = [sem_o0, sem_o1]

    def idx_start(col, h):
        return pltpu.async_copy(
            ainT_hbm.at[col, pl.ds(h * _BC, _BC)], idx_v.at[h], sem_i[h])

    def out_start(row, h):
        return pltpu.async_copy(
            idx_v.at[h], outT_hbm.at[row, w, pl.ds(h * _BC, _BC)], sem_o[h])

    def gather_half(h):
        @plsc.parallel_loop(0, _BC // 16, unroll=8)
        def _(g):
            sl = pl.ds(g * 16, 16)
            iv = plsc.bitcast(idx_v[h, sl], jnp.int32)
            idx_v[h, sl] = plsc.load_gather(row_v, [iv])

    def reg_half(h, ws, bs):
        @plsc.parallel_loop(0, _BC // 16, unroll=8)
        def _(g):
            sl = pl.ds(g * 16, 16)
            xf = plsc.bitcast(idx_v[h, sl], jnp.int32).astype(jnp.float32)
            idx_v[h, sl] = xf * ws + bs

    ic = [idx_start(_NUM_REG, 0), idx_start(_NUM_REG, 1)]
    oc = [None, None]
    for i in range(_NUM_CAT):
        pltpu.sync_copy(tableT_hbm.at[i, w], row_v)
        if oc[1] is not None:
            oc[1].wait()
            ic[1] = idx_start(_NUM_REG + i, 1)
        ic[0].wait()
        gather_half(0)
        oc[0] = out_start(i, 0)
        ic[1].wait()
        gather_half(1)
        oc[1] = out_start(i, 1)
        oc[0].wait()
        if i + 1 < _NUM_CAT:
            ic[0] = idx_start(_NUM_REG + i + 1, 0)

    oc[1].wait()
    ic[1] = idx_start(0, 1)
    ic[0] = idx_start(0, 0)
    for j in range(_NUM_REG):
        sel = jnp.full((16,), j * _D, jnp.int32) + w
        ws = plsc.load_gather(regw_v, [sel])
        bs = plsc.load_gather(regb_v, [sel])
        ic[0].wait()
        reg_half(0, ws, bs)
        oc[0] = out_start(_NUM_CAT + j, 0)
        ic[1].wait()
        reg_half(1, ws, bs)
        oc[1] = out_start(_NUM_CAT + j, 1)
        if j + 1 < _NUM_REG:
            oc[0].wait()
            ic[0] = idx_start(j + 1, 0)
            oc[1].wait()
            ic[1] = idx_start(j + 1, 1)
    oc[0].wait()
    oc[1].wait()


_sc_call = pl.kernel(
    _body,
    out_type=jax.ShapeDtypeStruct((_NCOLS, _D, _B), jnp.float32),
    mesh=plsc.VectorSubcoreMesh(core_axis_name="c", subcore_axis_name="s"),
    scratch_types=[
        pltpu.VMEM((_VOCAB,), jnp.float32),
        pltpu.VMEM((2, _BC), jnp.float32),
        pltpu.VMEM((_NUM_REG * _D,), jnp.float32),
        pltpu.VMEM((_NUM_REG * _D,), jnp.float32),
        pltpu.SemaphoreType.DMA,
        pltpu.SemaphoreType.DMA,
        pltpu.SemaphoreType.DMA,
        pltpu.SemaphoreType.DMA,
    ],
    compiler_params=pltpu.CompilerParams(
        needs_layout_passes=False, use_tc_tiling_on_sc=True),
)


@jax.jit
def kernel(all_inputs, emb_tables, reg_W, reg_b):
    ain_f = jax.lax.bitcast_convert_type(all_inputs, jnp.float32)
    ainT = ain_f.T
    tableT = emb_tables.transpose(0, 2, 1)
    outT = _sc_call(ainT, tableT, reg_W.reshape(-1), reg_b.reshape(-1))
    return outT.transpose(2, 0, 1)
